# baseline (speedup 1.0000x reference)
import functools

import jax
import jax.numpy as jnp
from jax import lax
from jax.experimental import pallas as pl
from jax.experimental.pallas import tpu as pltpu
from jax.experimental.pallas import tpu_sc as plsc

N = 10000
E = 160000
D = 128
PD = 16
F = D + PD
CNT_LANE = 8
CH = 128
NCH = E // CH
NC = 2
NS = 16
NW = NC * NS
STRIPE = N // NS
DT = 0.001
DH = 0.001

BE = 2000
BN = 2000

_mesh = plsc.VectorSubcoreMesh(core_axis_name="c", subcore_axis_name="s")


def _make_gather(nch):
    ne = nch * CH

    @functools.partial(
        pl.kernel,
        out_type=(
            jax.ShapeDtypeStruct((ne, D), jnp.float32),
            jax.ShapeDtypeStruct((ne, D), jnp.float32),
            jax.ShapeDtypeStruct((ne, PD), jnp.float32),
        ),
        mesh=_mesh,
        scratch_types=[
            pltpu.VMEM((CH,), jnp.int32),
            pltpu.VMEM((CH,), jnp.int32),
            pltpu.VMEM((CH, D), jnp.float32),
            pltpu.VMEM((CH, D), jnp.float32),
            pltpu.VMEM((CH, PD), jnp.float32),
            pltpu.VMEM((CH, PD), jnp.float32),
            pltpu.SemaphoreType.DMA,
            pltpu.SemaphoreType.DMA,
            pltpu.SemaphoreType.DMA,
            pltpu.SemaphoreType.DMA,
        ],
        compiler_params=pltpu.CompilerParams(use_tc_tiling_on_sc=False),
    )
    def _g(a_hbm, b_hbm, posp_hbm, row_hbm, col_hbm, ar_hbm, bc_hbm,
           pd_hbm, idxr, idxc, av, bv, prv, pcv, s0, s1, s2, s3):
        wid = lax.axis_index("s") * NC + lax.axis_index("c")
        nck = (nch - wid + NW - 1) // NW

        def chunk(k, carry):
            j = wid + k * NW
            base = j * CH
            pltpu.sync_copy(row_hbm.at[j], idxr)
            pltpu.sync_copy(col_hbm.at[j], idxc)
            cp0 = pltpu.async_copy(a_hbm.at[idxr], av, s0)
            cp1 = pltpu.async_copy(b_hbm.at[idxc], bv, s1)
            cp2 = pltpu.async_copy(posp_hbm.at[idxr], prv, s2)
            cp3 = pltpu.async_copy(posp_hbm.at[idxc], pcv, s3)
            cp2.wait()
            cp3.wait()

            def drow(i, c2):
                prv[i, :] = prv[i, :] - pcv[i, :]
                return c2

            lax.fori_loop(0, CH, drow, 0, unroll=4)
            cp0.wait()
            cp1.wait()
            pltpu.sync_copy(av, ar_hbm.at[pl.ds(base, CH)])
            pltpu.sync_copy(bv, bc_hbm.at[pl.ds(base, CH)])
            pltpu.sync_copy(prv, pd_hbm.at[pl.ds(base, CH)])
            return carry

        lax.fori_loop(0, nck, chunk, 0)

    return _g


NSPLIT = 5
_gather_part = _make_gather(NCH // NSPLIT)


def _make_scatter(nch):
    @functools.partial(
        pl.kernel,
        out_type=(
            jax.ShapeDtypeStruct((NC, N, D), jnp.float32),
            jax.ShapeDtypeStruct((NC, N, PD), jnp.float32),
        ),
        mesh=_mesh,
        scratch_types=[
            pltpu.VMEM((CH, D), jnp.float32),
            pltpu.VMEM((CH, PD), jnp.float32),
            pltpu.VMEM((CH,), jnp.int32),
            pltpu.VMEM_SHARED((N, D), jnp.float32),
            pltpu.VMEM_SHARED((N, PD), jnp.float32),
        ],
        compiler_params=pltpu.CompilerParams(use_tc_tiling_on_sc=False),
    )
    def _s(e_hbm, tr_hbm, row_hbm, z_hbm, zt_hbm, parts_hbm, parts2_hbm,
           ev, trv, idxv, acc, acc2):
        cid = lax.axis_index("c")
        sid = lax.axis_index("s")
        wid = sid * NC + cid
        pltpu.sync_copy(z_hbm, acc.at[pl.ds(sid * STRIPE, STRIPE)])
        pltpu.sync_copy(zt_hbm, acc2.at[pl.ds(sid * STRIPE, STRIPE)])
        plsc.subcore_barrier()

        nck = (nch - wid + NW - 1) // NW

        def chunk(k, carry):
            j = wid + k * NW
            pltpu.sync_copy(row_hbm.at[j], idxv)
            pltpu.sync_copy(e_hbm.at[pl.ds(j * CH, CH)], ev)
            pltpu.sync_copy(tr_hbm.at[pl.ds(j * CH, CH)], trv)
            pltpu.sync_copy(ev, acc.at[idxv], add=True)
            pltpu.sync_copy(trv, acc2.at[idxv], add=True)
            return carry

        lax.fori_loop(0, nck, chunk, 0)
        plsc.subcore_barrier()
        pltpu.sync_copy(acc.at[pl.ds(sid * STRIPE, STRIPE)],
                        parts_hbm.at[cid, pl.ds(sid * STRIPE, STRIPE)])
        pltpu.sync_copy(acc2.at[pl.ds(sid * STRIPE, STRIPE)],
                        parts2_hbm.at[cid, pl.ds(sid * STRIPE, STRIPE)])

    return _s


_scatter_part = _make_scatter(NCH // NSPLIT)


def _prep_body(h, W1a, W1b, b1, a_out, b_out):
    hb = h[...].astype(jnp.bfloat16)
    a_out[...] = (jnp.dot(hb, W1a[...], preferred_element_type=jnp.float32)
                  + b1[...])
    b_out[...] = jnp.dot(hb, W1b[...], preferred_element_type=jnp.float32)


def _tc_prep(h, W1a, W1b, b1):
    nb = N // BN
    wspec = lambda shape: pl.BlockSpec(shape, lambda i: (0, 0))
    return pl.pallas_call(
        _prep_body,
        grid=(nb,),
        in_specs=[
            pl.BlockSpec((BN, D), lambda i: (i, 0)),
            wspec((D, D)), wspec((D, D)), wspec((1, D)),
        ],
        out_specs=[
            pl.BlockSpec((BN, D), lambda i: (i, 0)),
            pl.BlockSpec((BN, D), lambda i: (i, 0)),
        ],
        out_shape=[
            jax.ShapeDtypeStruct((N, D), jnp.float32),
            jax.ShapeDtypeStruct((N, D), jnp.float32),
        ],
        compiler_params=pltpu.CompilerParams(
            dimension_semantics=("arbitrary",)),
    )(h, W1a, W1b, b1)


def _edge_body(ar, bc, pd, w1r, W2, b2, Wc1, bc1, Wc2, out_e, out_tr):
    f32 = jnp.float32
    bf = jnp.bfloat16
    pdv = pd[...]
    radial = jnp.sum(pdv * pdv, axis=1, keepdims=True)
    x = ar[...] + bc[...] + radial * w1r[...]
    x = x * jax.nn.sigmoid(x)
    x = jnp.dot(x.astype(bf), W2[...], preferred_element_type=f32) + b2[...]
    e = x * jax.nn.sigmoid(x)
    eb = e.astype(bf)
    y = jnp.dot(eb, Wc1[...], preferred_element_type=f32) + bc1[...]
    y = y * jax.nn.sigmoid(y)
    cw = y @ Wc2[...]
    tr = jnp.clip(pdv * cw, -100.0, 100.0)
    lane = lax.broadcasted_iota(jnp.int32, (1, PD), 1)
    tr = jnp.where(lane == CNT_LANE, 1.0, tr)
    out_e[...] = e
    out_tr[...] = tr


def _tc_edge(ar, bc, pd, w1r, W2, b2, Wc1, bc1, Wc2):
    ne = ar.shape[0]
    nb = ne // BE
    wspec = lambda shape: pl.BlockSpec(shape, lambda i: (0, 0))
    return pl.pallas_call(
        _edge_body,
        grid=(nb,),
        in_specs=[
            pl.BlockSpec((BE, D), lambda i: (i, 0)),
            pl.BlockSpec((BE, D), lambda i: (i, 0)),
            pl.BlockSpec((BE, PD), lambda i: (i, 0)),
            wspec((1, D)),
            wspec((D, D)), wspec((1, D)),
            wspec((D, D)), wspec((1, D)), wspec((D, 1)),
        ],
        out_specs=[
            pl.BlockSpec((BE, D), lambda i: (i, 0)),
            pl.BlockSpec((BE, PD), lambda i: (i, 0)),
        ],
        out_shape=[
            jax.ShapeDtypeStruct((ne, D), jnp.float32),
            jax.ShapeDtypeStruct((ne, PD), jnp.float32),
        ],
        compiler_params=pltpu.CompilerParams(
            dimension_semantics=("arbitrary",)),
    )(ar, bc, pd, w1r, W2, b2, Wc1, bc1, Wc2)


def _node_body(h, g, velp, posp, parts, parts2,
               Wv1, bv1, Wv2, bv2,
               Wn1a, Wn1b, bn1, Wn2, bn2,
               h2, g2, velp2, posp2, ldj):
    agg = jnp.sum(parts[...], axis=0)
    st = jnp.sum(parts2[...], axis=0)
    lane = lax.broadcasted_iota(jnp.int32, (1, PD), 1)
    cnt = jnp.sum(jnp.where(lane == CNT_LANE, st, 0.0), axis=1, keepdims=True)
    force = st / jnp.clip(cnt, 1.0, None)
    force = jnp.where(lane < 3, force, 0.0)

    hv = h[...]
    sv = hv @ Wv1[...] + bv1[...]
    sv = sv * jax.nn.sigmoid(sv)
    sv = sv @ Wv2[...] + bv2[...]

    x = hv @ Wn1a[...] + agg @ Wn1b[...] + bn1[...]
    x = x * jax.nn.sigmoid(x)
    no = x @ Wn2[...] + bn2[...]

    vel_new = jnp.exp(sv) * velp[...] + force * DT
    posp2[...] = posp[...] + vel_new * DT
    velp2[...] = vel_new
    g_new = g[...] + no * DH
    g2[...] = g_new
    h2[...] = hv + g_new * DH

    @pl.when(pl.program_id(0) == 0)
    def _():
        ldj[...] = jnp.zeros_like(ldj)

    ldj[...] += jnp.sum(sv)


def _tc_node(h, g, velp, posp, parts, parts2,
             Wv1, bv1, Wv2, bv2, Wn1a, Wn1b, bn1, Wn2, bn2):
    np_ = parts.shape[0]
    nb = N // BN
    wspec = lambda shape: pl.BlockSpec(shape, lambda i: tuple(0 for _ in shape))
    return pl.pallas_call(
        _node_body,
        grid=(nb,),
        in_specs=[
            pl.BlockSpec((BN, D), lambda i: (i, 0)),
            pl.BlockSpec((BN, D), lambda i: (i, 0)),
            pl.BlockSpec((BN, PD), lambda i: (i, 0)),
            pl.BlockSpec((BN, PD), lambda i: (i, 0)),
            pl.BlockSpec((np_, BN, D), lambda i: (0, i, 0)),
            pl.BlockSpec((np_, BN, PD), lambda i: (0, i, 0)),
            wspec((D, D)), wspec((1, D)), wspec((D, 1)), wspec((1, 1)),
            wspec((D, D)), wspec((D, D)), wspec((1, D)),
            wspec((D, D)), wspec((1, D)),
        ],
        out_specs=[
            pl.BlockSpec((BN, D), lambda i: (i, 0)),
            pl.BlockSpec((BN, D), lambda i: (i, 0)),
            pl.BlockSpec((BN, PD), lambda i: (i, 0)),
            pl.BlockSpec((BN, PD), lambda i: (i, 0)),
            pl.BlockSpec((1, 1), lambda i: (0, 0)),
        ],
        out_shape=[
            jax.ShapeDtypeStruct((N, D), jnp.float32),
            jax.ShapeDtypeStruct((N, D), jnp.float32),
            jax.ShapeDtypeStruct((N, PD), jnp.float32),
            jax.ShapeDtypeStruct((N, PD), jnp.float32),
            jax.ShapeDtypeStruct((1, 1), jnp.float32),
        ],
        compiler_params=pltpu.CompilerParams(
            dimension_semantics=("arbitrary",)),
    )(h, g, velp, posp, parts, parts2,
      Wv1, bv1, Wv2, bv2, Wn1a, Wn1b, bn1, Wn2, bn2)


def kernel(h, pos, g, vel, edge_index, W_e1, b_e1, W_e2, b_e2, W_n1, b_n1,
           W_n2, b_n2, W_c1, b_c1, W_c2, W_v1, b_v1, W_v2, b_v2):
    bf = jnp.bfloat16
    NH = NCH // NSPLIT
    row2 = edge_index[0].reshape(NCH, CH)
    col2 = edge_index[1].reshape(NCH, CH)
    rowh = [row2[k * NH:(k + 1) * NH] for k in range(NSPLIT)]
    colh = [col2[k * NH:(k + 1) * NH] for k in range(NSPLIT)]
    zpad = jnp.zeros((N, PD - 3), jnp.float32)
    posp = jnp.concatenate([pos, zpad], axis=1)
    velp = jnp.concatenate([vel, zpad], axis=1)
    zz = jnp.zeros((STRIPE, D), jnp.float32)
    zzt = jnp.zeros((STRIPE, PD), jnp.float32)

    ldj = jnp.zeros((), jnp.float32)
    for i in range(2):
        W1a = W_e1[i, :D].astype(bf)
        W1b = W_e1[i, D:2 * D].astype(bf)
        w1r = W_e1[i, 2 * D:]
        b1 = b_e1[i].reshape(1, D)
        W2, b2 = W_e2[i].astype(bf), b_e2[i].reshape(1, D)
        Wc1, bc1, Wc2 = W_c1[i].astype(bf), b_c1[i].reshape(1, D), W_c2[i]
        Wv1, bv1 = W_v1[i], b_v1[i].reshape(1, D)
        Wv2, bv2 = W_v2[i], b_v2[i].reshape(1, 1)
        Wn1a, Wn1b = W_n1[i, :D], W_n1[i, D:]
        bn1, Wn2, bn2 = b_n1[i].reshape(1, D), W_n2[i], b_n2[i].reshape(1, D)

        av, bv_ = _tc_prep(h, W1a, W1b, b1)
        gs = [_gather_part(av, bv_, posp, rowh[k], colh[k])
              for k in range(NSPLIT)]
        es = [_tc_edge(gk[0], gk[1], gk[2], w1r, W2, b2, Wc1, bc1, Wc2)
              for gk in gs]
        ps = [_scatter_part(ek[0], ek[1], rowh[k], zz, zzt)
              for k, ek in enumerate(es)]
        pstack = jnp.concatenate([p for p, _ in ps], axis=0)
        qstack = jnp.concatenate([q for _, q in ps], axis=0)
        h, g, velp, posp, lds = _tc_node(h, g, velp, posp, pstack, qstack,
                                         Wv1, bv1, Wv2, bv2,
                                         Wn1a, Wn1b, bn1, Wn2, bn2)
        ldj = ldj + lds[0, 0]

    return (h, g, posp[:, :3], velp[:, :3], ldj)


# speedup vs baseline: 1.2096x; 1.2096x over previous
import functools

import jax
import jax.numpy as jnp
from jax import lax
from jax.experimental import pallas as pl
from jax.experimental.pallas import tpu as pltpu
from jax.experimental.pallas import tpu_sc as plsc

N = 10000
E = 160000
D = 128
PD = 16
F = D + PD
CNT_LANE = 8
CH = 128
NCH = E // CH
NC = 2
NS = 16
NW = NC * NS
STRIPE = N // NS
DT = 0.001
DH = 0.001

BE = 2000
BN = 2000

_mesh = plsc.VectorSubcoreMesh(core_axis_name="c", subcore_axis_name="s")


def _make_gather(nch):
    ne = nch * CH

    @functools.partial(
        pl.kernel,
        out_type=(
            jax.ShapeDtypeStruct((ne, D), jnp.float32),
            jax.ShapeDtypeStruct((ne, D), jnp.float32),
            jax.ShapeDtypeStruct((ne, PD), jnp.float32),
        ),
        mesh=_mesh,
        scratch_types=(
            [pltpu.VMEM((CH,), jnp.int32)] * 4
            + [pltpu.VMEM((CH, D), jnp.float32)] * 4
            + [pltpu.VMEM((CH, PD), jnp.float32)] * 4
            + [pltpu.SemaphoreType.DMA] * 14
        ),
        compiler_params=pltpu.CompilerParams(use_tc_tiling_on_sc=False),
    )
    def _g(a_hbm, b_hbm, posp_hbm, row_hbm, col_hbm, ar_hbm, bc_hbm,
           pd_hbm,
           idxr0, idxc0, idxr1, idxc1, av0, bv0, av1, bv1,
           prv0, pcv0, prv1, pcv1,
           s0, s1, s2, s3, s4, s5, s6, s7,
           w0, w1, w2, w3, w4, w5):
        wid = lax.axis_index("s") * NC + lax.axis_index("c")
        nck = (nch - wid + NW - 1) // NW
        sets = ((idxr0, idxc0, av0, bv0, prv0, pcv0, (s0, s1, s2, s3),
                 (w0, w1, w2)),
                (idxr1, idxc1, av1, bv1, prv1, pcv1, (s4, s5, s6, s7),
                 (w3, w4, w5)))

        def issue(j, st):
            idxr, idxc, av, bv, prv, pcv, ss, _ = st
            pltpu.sync_copy(row_hbm.at[j], idxr)
            pltpu.sync_copy(col_hbm.at[j], idxc)
            cps = (pltpu.async_copy(a_hbm.at[idxr], av, ss[0]),
                   pltpu.async_copy(b_hbm.at[idxc], bv, ss[1]),
                   pltpu.async_copy(posp_hbm.at[idxr], prv, ss[2]),
                   pltpu.async_copy(posp_hbm.at[idxc], pcv, ss[3]))
            return cps

        def finish(j, st, cps):
            _, _, av, bv, prv, pcv, _, ws = st
            base = j * CH
            cps[2].wait()
            cps[3].wait()

            def drow(i, c2):
                prv[i, :] = prv[i, :] - pcv[i, :]
                return c2

            lax.fori_loop(0, CH, drow, 0, unroll=4)
            cps[0].wait()
            cps[1].wait()
            wcs = (pltpu.async_copy(av, ar_hbm.at[pl.ds(base, CH)], ws[0]),
                   pltpu.async_copy(bv, bc_hbm.at[pl.ds(base, CH)], ws[1]),
                   pltpu.async_copy(prv, pd_hbm.at[pl.ds(base, CH)], ws[2]))
            return wcs

        def pair(kp, carry):
            j0 = wid + (2 * kp) * NW
            j1 = wid + (2 * kp + 1) * NW
            cps0 = issue(j0, sets[0])
            cps1 = issue(j1, sets[1])
            wcs0 = finish(j0, sets[0], cps0)
            wcs1 = finish(j1, sets[1], cps1)
            for wc in wcs0 + wcs1:
                wc.wait()
            return carry

        lax.fori_loop(0, nck // 2, pair, 0)

        @pl.when(nck % 2 == 1)
        def _tail():
            j = wid + (nck - 1) * NW
            cps = issue(j, sets[0])
            wcs = finish(j, sets[0], cps)
            for wc in wcs:
                wc.wait()

    return _g


NSPLIT = 2
_gather_part = _make_gather(NCH // NSPLIT)


def _make_scatter(nch):
    @functools.partial(
        pl.kernel,
        out_type=(
            jax.ShapeDtypeStruct((NC, N, D), jnp.float32),
            jax.ShapeDtypeStruct((NC, N, PD), jnp.float32),
        ),
        mesh=_mesh,
        scratch_types=[
            pltpu.VMEM((CH, D), jnp.float32),
            pltpu.VMEM((CH, PD), jnp.float32),
            pltpu.VMEM((CH,), jnp.int32),
            pltpu.VMEM_SHARED((N, D), jnp.float32),
            pltpu.VMEM_SHARED((N, PD), jnp.float32),
        ],
        compiler_params=pltpu.CompilerParams(use_tc_tiling_on_sc=False),
    )
    def _s(e_hbm, tr_hbm, row_hbm, z_hbm, zt_hbm, parts_hbm, parts2_hbm,
           ev, trv, idxv, acc, acc2):
        cid = lax.axis_index("c")
        sid = lax.axis_index("s")
        wid = sid * NC + cid
        pltpu.sync_copy(z_hbm, acc.at[pl.ds(sid * STRIPE, STRIPE)])
        pltpu.sync_copy(zt_hbm, acc2.at[pl.ds(sid * STRIPE, STRIPE)])
        plsc.subcore_barrier()

        nck = (nch - wid + NW - 1) // NW

        def chunk(k, carry):
            j = wid + k * NW
            pltpu.sync_copy(row_hbm.at[j], idxv)
            pltpu.sync_copy(e_hbm.at[pl.ds(j * CH, CH)], ev)
            pltpu.sync_copy(tr_hbm.at[pl.ds(j * CH, CH)], trv)
            pltpu.sync_copy(ev, acc.at[idxv], add=True)
            pltpu.sync_copy(trv, acc2.at[idxv], add=True)
            return carry

        lax.fori_loop(0, nck, chunk, 0)
        plsc.subcore_barrier()
        pltpu.sync_copy(acc.at[pl.ds(sid * STRIPE, STRIPE)],
                        parts_hbm.at[cid, pl.ds(sid * STRIPE, STRIPE)])
        pltpu.sync_copy(acc2.at[pl.ds(sid * STRIPE, STRIPE)],
                        parts2_hbm.at[cid, pl.ds(sid * STRIPE, STRIPE)])

    return _s


_scatter_part = _make_scatter(NCH // NSPLIT)


def _prep_body(h, W1a, W1b, b1, a_out, b_out):
    hb = h[...].astype(jnp.bfloat16)
    a_out[...] = (jnp.dot(hb, W1a[...], preferred_element_type=jnp.float32)
                  + b1[...])
    b_out[...] = jnp.dot(hb, W1b[...], preferred_element_type=jnp.float32)


def _tc_prep(h, W1a, W1b, b1):
    nb = N // BN
    wspec = lambda shape: pl.BlockSpec(shape, lambda i: (0, 0))
    return pl.pallas_call(
        _prep_body,
        grid=(nb,),
        in_specs=[
            pl.BlockSpec((BN, D), lambda i: (i, 0)),
            wspec((D, D)), wspec((D, D)), wspec((1, D)),
        ],
        out_specs=[
            pl.BlockSpec((BN, D), lambda i: (i, 0)),
            pl.BlockSpec((BN, D), lambda i: (i, 0)),
        ],
        out_shape=[
            jax.ShapeDtypeStruct((N, D), jnp.float32),
            jax.ShapeDtypeStruct((N, D), jnp.float32),
        ],
        compiler_params=pltpu.CompilerParams(
            dimension_semantics=("arbitrary",)),
    )(h, W1a, W1b, b1)


def _edge_body(ar, bc, pd, w1r, W2, b2, Wc1, bc1, Wc2, out_e, out_tr):
    f32 = jnp.float32
    bf = jnp.bfloat16
    pdv = pd[...]
    radial = jnp.sum(pdv * pdv, axis=1, keepdims=True)
    x = ar[...] + bc[...] + radial * w1r[...]
    x = x * jax.nn.sigmoid(x)
    x = jnp.dot(x.astype(bf), W2[...], preferred_element_type=f32) + b2[...]
    e = x * jax.nn.sigmoid(x)
    eb = e.astype(bf)
    y = jnp.dot(eb, Wc1[...], preferred_element_type=f32) + bc1[...]
    y = y * jax.nn.sigmoid(y)
    cw = y @ Wc2[...]
    tr = jnp.clip(pdv * cw, -100.0, 100.0)
    lane = lax.broadcasted_iota(jnp.int32, (1, PD), 1)
    tr = jnp.where(lane == CNT_LANE, 1.0, tr)
    out_e[...] = e
    out_tr[...] = tr


def _tc_edge(ar, bc, pd, w1r, W2, b2, Wc1, bc1, Wc2):
    ne = ar.shape[0]
    nb = ne // BE
    wspec = lambda shape: pl.BlockSpec(shape, lambda i: (0, 0))
    return pl.pallas_call(
        _edge_body,
        grid=(nb,),
        in_specs=[
            pl.BlockSpec((BE, D), lambda i: (i, 0)),
            pl.BlockSpec((BE, D), lambda i: (i, 0)),
            pl.BlockSpec((BE, PD), lambda i: (i, 0)),
            wspec((1, D)),
            wspec((D, D)), wspec((1, D)),
            wspec((D, D)), wspec((1, D)), wspec((D, 1)),
        ],
        out_specs=[
            pl.BlockSpec((BE, D), lambda i: (i, 0)),
            pl.BlockSpec((BE, PD), lambda i: (i, 0)),
        ],
        out_shape=[
            jax.ShapeDtypeStruct((ne, D), jnp.float32),
            jax.ShapeDtypeStruct((ne, PD), jnp.float32),
        ],
        compiler_params=pltpu.CompilerParams(
            dimension_semantics=("arbitrary",)),
    )(ar, bc, pd, w1r, W2, b2, Wc1, bc1, Wc2)


def _node_body(h, g, velp, posp, parts, parts2,
               Wv1, bv1, Wv2, bv2,
               Wn1a, Wn1b, bn1, Wn2, bn2,
               h2, g2, velp2, posp2, ldj):
    agg = jnp.sum(parts[...], axis=0)
    st = jnp.sum(parts2[...], axis=0)
    lane = lax.broadcasted_iota(jnp.int32, (1, PD), 1)
    cnt = jnp.sum(jnp.where(lane == CNT_LANE, st, 0.0), axis=1, keepdims=True)
    force = st / jnp.clip(cnt, 1.0, None)
    force = jnp.where(lane < 3, force, 0.0)

    hv = h[...]
    sv = hv @ Wv1[...] + bv1[...]
    sv = sv * jax.nn.sigmoid(sv)
    sv = sv @ Wv2[...] + bv2[...]

    x = hv @ Wn1a[...] + agg @ Wn1b[...] + bn1[...]
    x = x * jax.nn.sigmoid(x)
    no = x @ Wn2[...] + bn2[...]

    vel_new = jnp.exp(sv) * velp[...] + force * DT
    posp2[...] = posp[...] + vel_new * DT
    velp2[...] = vel_new
    g_new = g[...] + no * DH
    g2[...] = g_new
    h2[...] = hv + g_new * DH

    @pl.when(pl.program_id(0) == 0)
    def _():
        ldj[...] = jnp.zeros_like(ldj)

    ldj[...] += jnp.sum(sv)


def _tc_node(h, g, velp, posp, parts, parts2,
             Wv1, bv1, Wv2, bv2, Wn1a, Wn1b, bn1, Wn2, bn2):
    np_ = parts.shape[0]
    nb = N // BN
    wspec = lambda shape: pl.BlockSpec(shape, lambda i: tuple(0 for _ in shape))
    return pl.pallas_call(
        _node_body,
        grid=(nb,),
        in_specs=[
            pl.BlockSpec((BN, D), lambda i: (i, 0)),
            pl.BlockSpec((BN, D), lambda i: (i, 0)),
            pl.BlockSpec((BN, PD), lambda i: (i, 0)),
            pl.BlockSpec((BN, PD), lambda i: (i, 0)),
            pl.BlockSpec((np_, BN, D), lambda i: (0, i, 0)),
            pl.BlockSpec((np_, BN, PD), lambda i: (0, i, 0)),
            wspec((D, D)), wspec((1, D)), wspec((D, 1)), wspec((1, 1)),
            wspec((D, D)), wspec((D, D)), wspec((1, D)),
            wspec((D, D)), wspec((1, D)),
        ],
        out_specs=[
            pl.BlockSpec((BN, D), lambda i: (i, 0)),
            pl.BlockSpec((BN, D), lambda i: (i, 0)),
            pl.BlockSpec((BN, PD), lambda i: (i, 0)),
            pl.BlockSpec((BN, PD), lambda i: (i, 0)),
            pl.BlockSpec((1, 1), lambda i: (0, 0)),
        ],
        out_shape=[
            jax.ShapeDtypeStruct((N, D), jnp.float32),
            jax.ShapeDtypeStruct((N, D), jnp.float32),
            jax.ShapeDtypeStruct((N, PD), jnp.float32),
            jax.ShapeDtypeStruct((N, PD), jnp.float32),
            jax.ShapeDtypeStruct((1, 1), jnp.float32),
        ],
        compiler_params=pltpu.CompilerParams(
            dimension_semantics=("arbitrary",)),
    )(h, g, velp, posp, parts, parts2,
      Wv1, bv1, Wv2, bv2, Wn1a, Wn1b, bn1, Wn2, bn2)


def kernel(h, pos, g, vel, edge_index, W_e1, b_e1, W_e2, b_e2, W_n1, b_n1,
           W_n2, b_n2, W_c1, b_c1, W_c2, W_v1, b_v1, W_v2, b_v2):
    bf = jnp.bfloat16
    NH = NCH // NSPLIT
    row2 = edge_index[0].reshape(NCH, CH)
    col2 = edge_index[1].reshape(NCH, CH)
    rowh = [row2[k * NH:(k + 1) * NH] for k in range(NSPLIT)]
    colh = [col2[k * NH:(k + 1) * NH] for k in range(NSPLIT)]
    zpad = jnp.zeros((N, PD - 3), jnp.float32)
    posp = jnp.concatenate([pos, zpad], axis=1)
    velp = jnp.concatenate([vel, zpad], axis=1)
    zz = jnp.zeros((STRIPE, D), jnp.float32)
    zzt = jnp.zeros((STRIPE, PD), jnp.float32)

    ldj = jnp.zeros((), jnp.float32)
    for i in range(2):
        W1a = W_e1[i, :D].astype(bf)
        W1b = W_e1[i, D:2 * D].astype(bf)
        w1r = W_e1[i, 2 * D:]
        b1 = b_e1[i].reshape(1, D)
        W2, b2 = W_e2[i].astype(bf), b_e2[i].reshape(1, D)
        Wc1, bc1, Wc2 = W_c1[i].astype(bf), b_c1[i].reshape(1, D), W_c2[i]
        Wv1, bv1 = W_v1[i], b_v1[i].reshape(1, D)
        Wv2, bv2 = W_v2[i], b_v2[i].reshape(1, 1)
        Wn1a, Wn1b = W_n1[i, :D], W_n1[i, D:]
        bn1, Wn2, bn2 = b_n1[i].reshape(1, D), W_n2[i], b_n2[i].reshape(1, D)

        av, bv_ = _tc_prep(h, W1a, W1b, b1)
        gs = [_gather_part(av, bv_, posp, rowh[k], colh[k])
              for k in range(NSPLIT)]
        es = [_tc_edge(gk[0], gk[1], gk[2], w1r, W2, b2, Wc1, bc1, Wc2)
              for gk in gs]
        ps = [_scatter_part(ek[0], ek[1], rowh[k], zz, zzt)
              for k, ek in enumerate(es)]
        pstack = jnp.concatenate([p for p, _ in ps], axis=0)
        qstack = jnp.concatenate([q for _, q in ps], axis=0)
        h, g, velp, posp, lds = _tc_node(h, g, velp, posp, pstack, qstack,
                                         Wv1, bv1, Wv2, bv2,
                                         Wn1a, Wn1b, bn1, Wn2, bn2)
        ldj = ldj + lds[0, 0]

    return (h, g, posp[:, :3], velp[:, :3], ldj)
